# baseline (speedup 1.0000x reference)
import functools

import jax
import jax.numpy as jnp
from jax import lax
from jax.experimental import pallas as pl
from jax.experimental.pallas import tpu as pltpu
from jax.experimental.pallas import tpu_sc as plsc

EMBED_DIM = 32
BATCH = 4096
HIST = 50
TOTAL = BATCH * HIST

NUM_CORES = 2
NUM_SUBCORES = 16
NW = NUM_CORES * NUM_SUBCORES
BPW = TOTAL // NW
NCHUNK = 4
CHUNK = BPW // NCHUNK


def _gather_kernel(idx_hbm, table_hbm, out_hbm, idx_v, rows_v, sem):
    wid = lax.axis_index("s") * NUM_CORES + lax.axis_index("c")
    base = wid * BPW
    pltpu.sync_copy(idx_hbm.at[wid], idx_v)

    copies = [None, None]
    copies[0] = pltpu.async_copy(table_hbm.at[idx_v.at[0]], rows_v.at[0], sem)
    for j in range(1, NCHUNK):
        copies[j % 2] = pltpu.async_copy(
            table_hbm.at[idx_v.at[j]], rows_v.at[j % 2], sem)
        copies[(j - 1) % 2].wait()
        pltpu.sync_copy(rows_v.at[(j - 1) % 2],
                        out_hbm.at[pl.ds(base + (j - 1) * CHUNK, CHUNK)])
    copies[(NCHUNK - 1) % 2].wait()
    pltpu.sync_copy(rows_v.at[(NCHUNK - 1) % 2],
                    out_hbm.at[pl.ds(base + (NCHUNK - 1) * CHUNK, CHUNK)])


@jax.jit
def kernel(inputs, embeddings):
    idx = inputs.reshape(-1).astype(jnp.int32).reshape(NW, NCHUNK, CHUNK)
    mesh = plsc.VectorSubcoreMesh(core_axis_name="c", subcore_axis_name="s")
    run = functools.partial(
        pl.kernel,
        mesh=mesh,
        out_type=jax.ShapeDtypeStruct((TOTAL, EMBED_DIM), jnp.float32),
        scratch_types=[
            pltpu.VMEM((NCHUNK, CHUNK), jnp.int32),
            pltpu.VMEM((2, CHUNK, EMBED_DIM), jnp.float32),
            pltpu.SemaphoreType.DMA,
        ],
        compiler_params=pltpu.CompilerParams(use_tc_tiling_on_sc=False),
    )(_gather_kernel)
    out = run(idx, embeddings)
    return out.reshape(BATCH, HIST, EMBED_DIM)


# speedup vs baseline: 1.7341x; 1.7341x over previous
import functools

import jax
import jax.numpy as jnp
from jax import lax
from jax.experimental import pallas as pl
from jax.experimental.pallas import tpu as pltpu
from jax.experimental.pallas import tpu_sc as plsc

EMBED_DIM = 32
BATCH = 4096
HIST = 50

NUM_CORES = 2
NUM_SUBCORES = 16
NW = NUM_CORES * NUM_SUBCORES
BPW = BATCH // NW
CB = 8
NCHUNK = BPW // CB


def _issue_row_gathers(table_hbm, rows_v, idx_c, slot, bi, gsem):
    vecs = [
        idx_c[bi, pl.ds(0, 16)],
        idx_c[bi, pl.ds(16, 16)],
        idx_c[bi, pl.ds(32, 16)],
        idx_c[bi, pl.ds(34, 16)],
    ]
    for h in range(HIST):
        if h < 48:
            i = vecs[h // 16][h % 16]
        else:
            i = vecs[3][h - 34]
        pltpu.async_copy(table_hbm.at[i], rows_v.at[slot, bi, h], gsem)


def _gather_kernel(idx_hbm, table_hbm, out_hbm, idx_v, rows_v,
                   gsem, wsem, isem):
    wid = lax.axis_index("s") * NUM_CORES + lax.axis_index("c")
    b0 = wid * BPW
    pltpu.sync_copy(idx_hbm.at[pl.ds(b0, CB)], idx_v.at[0])

    def chunk_body(g, _):
        slot = lax.rem(g, 2)
        bb = b0 + g * CB

        @pl.when(g >= 2)
        def _():
            pltpu.make_async_copy(
                rows_v.at[slot], out_hbm.at[pl.ds(bb, CB)], wsem).wait()

        @pl.when(g >= 1)
        def _():
            pltpu.make_async_copy(
                idx_hbm.at[pl.ds(bb, CB)], idx_v.at[slot], isem).wait()

        @pl.when(g <= NCHUNK - 2)
        def _():
            pltpu.async_copy(
                idx_hbm.at[pl.ds(bb + CB, CB)], idx_v.at[1 - slot], isem)

        def body(bi, _):
            _issue_row_gathers(table_hbm, rows_v, idx_v.at[slot],
                               slot, bi, gsem)
            return ()

        lax.fori_loop(0, CB, body, ())
        pltpu.make_async_copy(
            out_hbm.at[pl.ds(bb, CB)], rows_v.at[slot], gsem).wait()
        pltpu.async_copy(rows_v.at[slot], out_hbm.at[pl.ds(bb, CB)], wsem)
        return ()

    lax.fori_loop(0, NCHUNK, chunk_body, ())
    pltpu.make_async_copy(
        rows_v.at[0], out_hbm.at[pl.ds(b0, CB)], wsem).wait()
    pltpu.make_async_copy(
        rows_v.at[1], out_hbm.at[pl.ds(b0, CB)], wsem).wait()


@jax.jit
def kernel(inputs, embeddings):
    idx = inputs.astype(jnp.int32)
    mesh = plsc.VectorSubcoreMesh(core_axis_name="c", subcore_axis_name="s")
    run = functools.partial(
        pl.kernel,
        mesh=mesh,
        out_type=jax.ShapeDtypeStruct((BATCH, HIST, EMBED_DIM), jnp.float32),
        scratch_types=[
            pltpu.VMEM((2, CB, HIST), jnp.int32),
            pltpu.VMEM((2, CB, HIST, EMBED_DIM), jnp.float32),
            pltpu.SemaphoreType.DMA,
            pltpu.SemaphoreType.DMA,
            pltpu.SemaphoreType.DMA,
        ],
    )(_gather_kernel)
    return run(idx, embeddings)
